# Initial kernel scaffold; baseline (speedup 1.0000x reference)
import jax
import jax.numpy as jnp
from jax.experimental import pallas as pl


def kernel(x, embeddings):
    raise NotImplementedError("write your pallas kernel here")


# speedup vs baseline: 1.4117x; 1.4117x over previous
import functools

import jax
import jax.numpy as jnp
from jax import lax
from jax.experimental import pallas as pl
from jax.experimental.pallas import tpu as pltpu
from jax.experimental.pallas import tpu_sc as plsc

NE = 8192
D = 256
NT = 8192
MB = 1024
NB = 1024
NW = 32
BPW = NT // NW
BETA = 0.25
EPS = 1e-10


def _argmin_body(x_ref, e_ref, idx_ref, min_ref):
    j = pl.program_id(1)
    xb = x_ref[...]
    eb = e_ref[...]
    x2 = jnp.sum(xb * xb, axis=1)[:, None]
    e2 = jnp.sum(eb * eb, axis=1)[None, :]
    s = lax.dot_general(xb, eb, (((1,), (1,)), ((), ())),
                        preferred_element_type=jnp.float32)
    d = (x2 + e2) - 2.0 * s
    m = jnp.min(d, axis=1, keepdims=True)
    cols = lax.broadcasted_iota(jnp.int32, d.shape, 1)
    loc = jnp.min(jnp.where(d == m, cols, NB), axis=1, keepdims=True)
    gidx = loc + j * NB

    @pl.when(j == 0)
    def _():
        min_ref[...] = m
        idx_ref[...] = gidx

    @pl.when(j != 0)
    def _():
        prev = min_ref[...]
        better = m < prev
        min_ref[...] = jnp.where(better, m, prev)
        idx_ref[...] = jnp.where(better, gidx, idx_ref[...])


_argmin_call = pl.pallas_call(
    _argmin_body,
    grid=(NT // MB, NE // NB),
    in_specs=[
        pl.BlockSpec((MB, D), lambda i, j: (i, 0)),
        pl.BlockSpec((NB, D), lambda i, j: (j, 0)),
    ],
    out_specs=[
        pl.BlockSpec((MB, 1), lambda i, j: (i, 0)),
        pl.BlockSpec((MB, 1), lambda i, j: (i, 0)),
    ],
    out_shape=[
        jax.ShapeDtypeStruct((NT, 1), jnp.int32),
        jax.ShapeDtypeStruct((NT, 1), jnp.float32),
    ],
)


def _sc_body(idx_hbm, emb_hbm, quant_hbm, pcnt_hbm, idx_v, rows_v, cnt_v, sem):
    wid = lax.axis_index("s") * 2 + lax.axis_index("c")
    base = wid * BPW
    pltpu.sync_copy(idx_hbm.at[pl.ds(base, BPW)], idx_v)
    for c in range(BPW // 128):
        pltpu.async_copy(emb_hbm.at[idx_v.at[pl.ds(c * 128, 128)]],
                         rows_v.at[pl.ds(c * 128, 128)], sem).wait()
    pltpu.sync_copy(rows_v, quant_hbm.at[pl.ds(base, BPW)])

    def zero_body(i, carry):
        cnt_v[pl.ds(i * 16, 16)] = jnp.zeros((16,), jnp.int32)
        return carry

    lax.fori_loop(0, NE // 16, zero_body, 0)

    lanes = lax.broadcasted_iota(jnp.int32, (16,), 0)
    ones = jnp.ones((16,), jnp.int32)

    def bin_body(g, carry):
        v = idx_v[pl.ds(g * 16, 16)]
        for l in range(16):
            plsc.addupdate_scatter(cnt_v, [v], ones, mask=lanes == l)
        return carry

    lax.fori_loop(0, BPW // 16, bin_body, 0)
    pltpu.sync_copy(cnt_v, pcnt_hbm.at[wid])


@functools.cache
def _sc_call():
    return pl.kernel(
        _sc_body,
        out_type=(
            jax.ShapeDtypeStruct((NT, D), jnp.float32),
            jax.ShapeDtypeStruct((NW, NE), jnp.int32),
        ),
        mesh=plsc.VectorSubcoreMesh(core_axis_name="c", subcore_axis_name="s"),
        compiler_params=pltpu.CompilerParams(needs_layout_passes=False),
        scratch_types=[
            pltpu.VMEM((BPW,), jnp.int32),
            pltpu.VMEM((BPW, D), jnp.float32),
            pltpu.VMEM((NE,), jnp.int32),
            pltpu.SemaphoreType.DMA,
        ],
    )


def _stats_body(pc_ref, mv_ref, cnt_ref, loss_ref, perp_ref):
    counts = jnp.sum(pc_ref[...], axis=0)
    cnt_ref[...] = counts
    p = counts.astype(jnp.float32) / float(NT)
    ent = jnp.sum(p * jnp.log(p + EPS))
    perp_ref[...] = jnp.full((1, 1), jnp.exp(-ent), jnp.float32)
    m = jnp.sum(mv_ref[...]) / float(NT * D)
    loss_ref[...] = jnp.full((1, 1), BETA * m + m, jnp.float32)


_stats_call = pl.pallas_call(
    _stats_body,
    out_shape=[
        jax.ShapeDtypeStruct((64, 128), jnp.int32),
        jax.ShapeDtypeStruct((1, 1), jnp.float32),
        jax.ShapeDtypeStruct((1, 1), jnp.float32),
    ],
)


@jax.jit
def kernel(x, embeddings):
    xf = x.reshape(NT, D)
    idx2d, minv = _argmin_call(xf, embeddings)
    quant, pcnt = _sc_call()(idx2d.reshape(NT), embeddings)
    counts2d, loss, perp = _stats_call(pcnt.reshape(NW, 64, 128),
                                       minv.reshape(NT // MB, MB))
    return (quant.reshape(x.shape), perp.reshape(()), loss.reshape(()),
            counts2d.reshape(NE))
